# baseline (speedup 1.0000x reference)
import functools

import jax
import jax.numpy as jnp
from jax import lax
from jax.experimental import pallas as pl
from jax.experimental.pallas import tpu as pltpu
from jax.experimental.pallas import tpu_sc as plsc

_NUM_WORDS = 100000
_T = 32
_D = 64
_LANES = 16

_NC = 2
_NS = 16
_NW = _NC * _NS

_CH = 8
_HF = 4
_HT = _HF * _T


def _body(ids_hbm, toks_hbm, cnts_hbm, emb_hbm, idf_hbm, out_hbm,
          idf_v, ids_v, toks2, cnts2, idxf, emb2, out_v, sem_t, gsem_a, gsem_b):
    qpw = ids_v.shape[0] - 2 * _CH
    n_chunks = qpw // _CH
    wid = lax.axis_index("s") * _NC + lax.axis_index("c")
    base = wid * qpw

    pltpu.sync_copy(idf_hbm, idf_v)
    ids_v[pl.ds(qpw, 2 * _CH)] = jnp.zeros((2 * _CH,), jnp.int32)
    pltpu.sync_copy(ids_hbm.at[pl.ds(base, qpw)], ids_v.at[pl.ds(0, qpw)])

    def fire_toks(c, par):
        idx = ids_v.at[pl.ds(c * _CH, _CH)]
        cp_t = pltpu.async_copy(toks_hbm.at[idx], toks2.at[par], sem_t)
        cp_c = pltpu.async_copy(cnts_hbm.at[idx], cnts2.at[par], sem_t)
        return cp_t, cp_c

    def fire_emb(par, hoff, slot, gsem):
        for jj in range(_HF):
            for h in range(2):
                idxf[par, hoff, pl.ds(jj * _T + h * _LANES, _LANES)] = \
                    toks2[par, hoff * _HF + jj, pl.ds(h * _LANES, _LANES)]
        pltpu.async_copy(emb_hbm.at[idxf.at[par, hoff]], emb2.at[slot], gsem)

    def drain_emb(slot, gsem):
        pltpu.make_async_copy(emb_hbm.at[pl.ds(0, _HT)],
                              emb2.at[slot], gsem).wait()

    def compute4(par, qoff, slot):
        for jj in range(_HF):
            q = qoff + jj
            t0 = toks2[par, q, pl.ds(0, _LANES)]
            t1 = toks2[par, q, pl.ds(_LANES, _LANES)]
            w0 = cnts2[par, q, pl.ds(0, _LANES)] * plsc.load_gather(idf_v, [t0])
            w1 = cnts2[par, q, pl.ds(_LANES, _LANES)] * plsc.load_gather(idf_v, [t1])
            denom = jnp.maximum(jnp.sum(w0 + w1), 1e-6)
            scale = 1.0 / jnp.broadcast_to(denom, (_LANES,))

            accs = [jnp.zeros((_LANES,), jnp.float32) for _ in range(4)]
            hi_mask = jnp.full((_LANES,), -65536, jnp.int32)
            for half, wv in enumerate((w0, w1)):
                for tt in range(_LANES):
                    t = half * _LANES + tt
                    wt = jnp.broadcast_to(wv[tt], (_LANES,))
                    for blk in range(2):
                        v = emb2[slot, jj * _T + t, pl.ds(blk * _LANES, _LANES)]
                        lo = plsc.bitcast(v << 16, jnp.float32)
                        hi = plsc.bitcast(v & hi_mask, jnp.float32)
                        accs[2 * blk] = accs[2 * blk] + wt * lo
                        accs[2 * blk + 1] = accs[2 * blk + 1] + wt * hi
            lane2 = 2 * lax.iota(jnp.int32, _LANES)
            qvec = jnp.full((_LANES,), q, jnp.int32)
            for blk in range(2):
                plsc.store_scatter(out_v, [qvec, lane2 + 32 * blk],
                                   accs[2 * blk] * scale)
                plsc.store_scatter(out_v, [qvec, lane2 + 32 * blk + 1],
                                   accs[2 * blk + 1] * scale)

    cp_t, cp_c = fire_toks(0, 0)
    cp_t.wait()
    cp_c.wait()
    fire_emb(0, 0, 0, gsem_a)

    @pl.loop(0, n_chunks // 2)
    def _pair(cp):
        for p in range(2):
            c = cp * 2 + p
            cp_t, cp_c = fire_toks(c + 1, 1 - p)
            fire_emb(p, 1, 1, gsem_b)
            drain_emb(0, gsem_a)
            compute4(p, 0, 0)
            cp_t.wait()
            cp_c.wait()
            fire_emb(1 - p, 0, 0, gsem_a)
            drain_emb(1, gsem_b)
            compute4(p, _HF, 1)
            pltpu.sync_copy(out_v, out_hbm.at[pl.ds(base + c * _CH, _CH)])

    drain_emb(0, gsem_a)


def kernel(entity_id, entity_tokens, entity_counts, word_embeds, idf):
    b, l = entity_id.shape
    q = b * l
    qpw = q // _NW
    flat_ids = entity_id.reshape(q).astype(jnp.int32)

    mesh = plsc.VectorSubcoreMesh(core_axis_name="c", subcore_axis_name="s")
    run = functools.partial(
        pl.kernel,
        out_type=jax.ShapeDtypeStruct((q, _D), jnp.float32),
        mesh=mesh,
        compiler_params=pltpu.CompilerParams(
            needs_layout_passes=False, use_tc_tiling_on_sc=False),
        scratch_types=[
            pltpu.VMEM((_NUM_WORDS,), jnp.float32),
            pltpu.VMEM((qpw + 2 * _CH,), jnp.int32),
            pltpu.VMEM((2, _CH, _T), jnp.int32),
            pltpu.VMEM((2, _CH, _T), jnp.float32),
            pltpu.VMEM((2, 2, _HT), jnp.int32),
            pltpu.VMEM((2, _HT, _D // 2), jnp.int32),
            pltpu.VMEM((_CH, _D), jnp.float32),
            pltpu.SemaphoreType.DMA,
            pltpu.SemaphoreType.DMA,
            pltpu.SemaphoreType.DMA,
        ],
    )(_body)
    emb_packed = jax.lax.bitcast_convert_type(
        word_embeds.astype(jnp.bfloat16).reshape(_NUM_WORDS, _D // 2, 2),
        jnp.int32)
    out = run(flat_ids, entity_tokens, entity_counts, emb_packed, idf)
    return out.reshape(b, l, _D)


# speedup vs baseline: 1.4477x; 1.4477x over previous
import functools

import jax
import jax.numpy as jnp
from jax import lax
from jax.experimental import pallas as pl
from jax.experimental.pallas import tpu as pltpu
from jax.experimental.pallas import tpu_sc as plsc

_NUM_WORDS = 100000
_T = 32
_D = 64
_LANES = 16

_NC = 2
_NS = 16
_NW = _NC * _NS

_CH = 8
_HF = 4
_HT = _HF * _T


def _body(ids_hbm, toks_hbm, cnts_hbm, emb_hbm, idf_hbm, out_hbm,
          idf_v, ids_v, toks2, cnts2, idxf, emb2, out_v, sem_t, gsem_a, gsem_b):
    qpw = ids_v.shape[0] - 2 * _CH
    n_chunks = qpw // _CH
    wid = lax.axis_index("s") * _NC + lax.axis_index("c")
    base = wid * qpw

    pltpu.sync_copy(idf_hbm, idf_v)
    ids_v[pl.ds(qpw, 2 * _CH)] = jnp.zeros((2 * _CH,), jnp.int32)
    pltpu.sync_copy(ids_hbm.at[pl.ds(base, qpw)], ids_v.at[pl.ds(0, qpw)])

    def fire_toks(c, par):
        idx = ids_v.at[pl.ds(c * _CH, _CH)]
        cp_t = pltpu.async_copy(toks_hbm.at[idx], toks2.at[par], sem_t)
        cp_c = pltpu.async_copy(cnts_hbm.at[idx], cnts2.at[par], sem_t)
        return cp_t, cp_c

    def fire_emb(par, hoff, slot, gsem):
        for jj in range(_HF):
            for h in range(2):
                idxf[par, hoff, pl.ds(jj * _T + h * _LANES, _LANES)] = \
                    toks2[par, hoff * _HF + jj, pl.ds(h * _LANES, _LANES)]
        pltpu.async_copy(emb_hbm.at[idxf.at[par, hoff]], emb2.at[slot], gsem)

    def drain_emb(slot, gsem):
        pltpu.make_async_copy(emb_hbm.at[pl.ds(0, _HT)],
                              emb2.at[slot], gsem).wait()

    def compute4(par, qoff, slot):
        for jj in range(_HF):
            q = qoff + jj
            t0 = toks2[par, q, pl.ds(0, _LANES)]
            t1 = toks2[par, q, pl.ds(_LANES, _LANES)]
            w0 = cnts2[par, q, pl.ds(0, _LANES)] * plsc.load_gather(idf_v, [t0])
            w1 = cnts2[par, q, pl.ds(_LANES, _LANES)] * plsc.load_gather(idf_v, [t1])
            denom = jnp.maximum(jnp.sum(w0 + w1), 1e-6)
            scale = 1.0 / jnp.broadcast_to(denom, (_LANES,))

            accs = [jnp.zeros((_LANES,), jnp.float32) for _ in range(4)]
            hi_mask = jnp.full((_LANES,), -65536, jnp.int32)
            for half, wv in enumerate((w0, w1)):
                for tt in range(_LANES):
                    t = half * _LANES + tt
                    wt = jnp.broadcast_to(wv[tt], (_LANES,))
                    for blk in range(2):
                        v = plsc.bitcast(
                            emb2[slot, jj * _T + t, pl.ds(blk * 2 * _LANES, 2 * _LANES)],
                            jnp.int32)
                        lo = plsc.bitcast(v << 16, jnp.float32)
                        hi = plsc.bitcast(v & hi_mask, jnp.float32)
                        accs[2 * blk] = accs[2 * blk] + wt * lo
                        accs[2 * blk + 1] = accs[2 * blk + 1] + wt * hi
            lane2 = 2 * lax.iota(jnp.int32, _LANES)
            qvec = jnp.full((_LANES,), q, jnp.int32)
            for blk in range(2):
                plsc.store_scatter(out_v, [qvec, lane2 + 32 * blk],
                                   accs[2 * blk] * scale)
                plsc.store_scatter(out_v, [qvec, lane2 + 32 * blk + 1],
                                   accs[2 * blk + 1] * scale)

    cp_t, cp_c = fire_toks(0, 0)
    cp_t.wait()
    cp_c.wait()
    fire_emb(0, 0, 0, gsem_a)

    @pl.loop(0, n_chunks // 2)
    def _pair(cp):
        for p in range(2):
            c = cp * 2 + p
            cp_t, cp_c = fire_toks(c + 1, 1 - p)
            fire_emb(p, 1, 1, gsem_b)
            drain_emb(0, gsem_a)
            compute4(p, 0, 0)
            cp_t.wait()
            cp_c.wait()
            fire_emb(1 - p, 0, 0, gsem_a)
            drain_emb(1, gsem_b)
            compute4(p, _HF, 1)
            pltpu.sync_copy(out_v, out_hbm.at[pl.ds(base + c * _CH, _CH)])

    drain_emb(0, gsem_a)


def kernel(entity_id, entity_tokens, entity_counts, word_embeds, idf):
    b, l = entity_id.shape
    q = b * l
    qpw = q // _NW
    flat_ids = entity_id.reshape(q).astype(jnp.int32)

    mesh = plsc.VectorSubcoreMesh(core_axis_name="c", subcore_axis_name="s")
    run = functools.partial(
        pl.kernel,
        out_type=jax.ShapeDtypeStruct((q, _D), jnp.float32),
        mesh=mesh,
        compiler_params=pltpu.CompilerParams(
            needs_layout_passes=False, use_tc_tiling_on_sc=False),
        scratch_types=[
            pltpu.VMEM((_NUM_WORDS,), jnp.float32),
            pltpu.VMEM((qpw + 2 * _CH,), jnp.int32),
            pltpu.VMEM((2, _CH, _T), jnp.int32),
            pltpu.VMEM((2, _CH, _T), jnp.float32),
            pltpu.VMEM((2, 2, _HT), jnp.int32),
            pltpu.VMEM((2, _HT, _D), jnp.bfloat16),
            pltpu.VMEM((_CH, _D), jnp.float32),
            pltpu.SemaphoreType.DMA,
            pltpu.SemaphoreType.DMA,
            pltpu.SemaphoreType.DMA,
        ],
    )(_body)
    out = run(flat_ids, entity_tokens, entity_counts,
              word_embeds.astype(jnp.bfloat16), idf)
    return out.reshape(b, l, _D)


# speedup vs baseline: 1.4530x; 1.0037x over previous
import functools

import jax
import jax.numpy as jnp
from jax import lax
from jax.experimental import pallas as pl
from jax.experimental.pallas import tpu as pltpu
from jax.experimental.pallas import tpu_sc as plsc

_NUM_WORDS = 100000
_T = 32
_D = 64
_LANES = 16

_NC = 2
_NS = 16
_NW = _NC * _NS

_CH = 8
_HF = 4
_HT = _HF * _T


def _body(ids_hbm, toks_hbm, cnts_hbm, emb_hbm, idf_hbm, out_hbm,
          idf_v, ids_v, toks2, cnts2, idxf, emb2, out_v,
          sem_t0, sem_t1, gsem_a, gsem_b):
    qpw = ids_v.shape[0] - 2 * _CH
    n_chunks = qpw // _CH
    wid = lax.axis_index("s") * _NC + lax.axis_index("c")
    base = wid * qpw

    pltpu.sync_copy(idf_hbm, idf_v)
    ids_v[pl.ds(qpw, 2 * _CH)] = jnp.zeros((2 * _CH,), jnp.int32)
    pltpu.sync_copy(ids_hbm.at[pl.ds(base, qpw)], ids_v.at[pl.ds(0, qpw)])

    def fire_toks(c, sl, sem):
        idx = ids_v.at[pl.ds(c * _CH, _CH)]
        cp_t = pltpu.async_copy(toks_hbm.at[idx], toks2.at[sl], sem)
        cp_c = pltpu.async_copy(cnts_hbm.at[idx], cnts2.at[sl], sem)
        return cp_t, cp_c

    def wait_toks(sem):
        pltpu.make_async_copy(toks_hbm.at[pl.ds(0, _CH)],
                              toks2.at[0], sem).wait()
        pltpu.make_async_copy(cnts_hbm.at[pl.ds(0, _CH)],
                              cnts2.at[0], sem).wait()

    def fire_emb(sl, par, hoff, slot, gsem):
        for jj in range(_HF):
            for h in range(2):
                idxf[par, hoff, pl.ds(jj * _T + h * _LANES, _LANES)] = \
                    toks2[sl, hoff * _HF + jj, pl.ds(h * _LANES, _LANES)]
        pltpu.async_copy(emb_hbm.at[idxf.at[par, hoff]], emb2.at[slot], gsem)

    def drain_emb(slot, gsem):
        pltpu.make_async_copy(emb_hbm.at[pl.ds(0, _HT)],
                              emb2.at[slot], gsem).wait()

    def compute4(sl, qoff, slot):
        for jj in range(_HF):
            q = qoff + jj
            t0 = toks2[sl, q, pl.ds(0, _LANES)]
            t1 = toks2[sl, q, pl.ds(_LANES, _LANES)]
            w0 = cnts2[sl, q, pl.ds(0, _LANES)] * plsc.load_gather(idf_v, [t0])
            w1 = cnts2[sl, q, pl.ds(_LANES, _LANES)] * plsc.load_gather(idf_v, [t1])
            denom = jnp.maximum(jnp.sum(w0 + w1), 1e-6)
            scale = 1.0 / jnp.broadcast_to(denom, (_LANES,))

            accs = [jnp.zeros((_LANES,), jnp.float32) for _ in range(4)]
            hi_mask = jnp.full((_LANES,), -65536, jnp.int32)
            for half, wv in enumerate((w0, w1)):
                for tt in range(_LANES):
                    t = half * _LANES + tt
                    wt = jnp.broadcast_to(wv[tt], (_LANES,))
                    for blk in range(2):
                        v = plsc.bitcast(
                            emb2[slot, jj * _T + t, pl.ds(blk * 2 * _LANES, 2 * _LANES)],
                            jnp.int32)
                        lo = plsc.bitcast(v << 16, jnp.float32)
                        hi = plsc.bitcast(v & hi_mask, jnp.float32)
                        accs[2 * blk] = accs[2 * blk] + wt * lo
                        accs[2 * blk + 1] = accs[2 * blk + 1] + wt * hi
            lane2 = 2 * lax.iota(jnp.int32, _LANES)
            qvec = jnp.full((_LANES,), q, jnp.int32)
            for blk in range(2):
                plsc.store_scatter(out_v, [qvec, lane2 + 32 * blk],
                                   accs[2 * blk] * scale)
                plsc.store_scatter(out_v, [qvec, lane2 + 32 * blk + 1],
                                   accs[2 * blk + 1] * scale)

    cp_t, cp_c = fire_toks(0, 0, sem_t0)
    cp_t.wait()
    cp_c.wait()
    fire_toks(1, 1, sem_t1)
    fire_emb(0, 0, 0, 0, gsem_a)

    sems = (sem_t0, sem_t1)

    @pl.loop(0, n_chunks // 2)
    def _pair(cp):
        for p in range(2):
            c = cp * 2 + p
            slc = lax.rem(c, 4)
            fire_toks(c + 2, lax.rem(c + 2, 4), sems[p])
            fire_emb(slc, p, 1, 1, gsem_b)
            drain_emb(0, gsem_a)
            compute4(slc, 0, 0)
            wait_toks(sems[1 - p])
            fire_emb(lax.rem(c + 1, 4), 1 - p, 0, 0, gsem_a)
            drain_emb(1, gsem_b)
            compute4(slc, _HF, 1)
            pltpu.sync_copy(out_v, out_hbm.at[pl.ds(base + c * _CH, _CH)])

    drain_emb(0, gsem_a)
    wait_toks(sem_t1)


def kernel(entity_id, entity_tokens, entity_counts, word_embeds, idf):
    b, l = entity_id.shape
    q = b * l
    qpw = q // _NW
    flat_ids = entity_id.reshape(q).astype(jnp.int32)

    mesh = plsc.VectorSubcoreMesh(core_axis_name="c", subcore_axis_name="s")
    run = functools.partial(
        pl.kernel,
        out_type=jax.ShapeDtypeStruct((q, _D), jnp.float32),
        mesh=mesh,
        compiler_params=pltpu.CompilerParams(
            needs_layout_passes=False, use_tc_tiling_on_sc=False),
        scratch_types=[
            pltpu.VMEM((_NUM_WORDS,), jnp.float32),
            pltpu.VMEM((qpw + 2 * _CH,), jnp.int32),
            pltpu.VMEM((4, _CH, _T), jnp.int32),
            pltpu.VMEM((4, _CH, _T), jnp.float32),
            pltpu.VMEM((2, 2, _HT), jnp.int32),
            pltpu.VMEM((2, _HT, _D), jnp.bfloat16),
            pltpu.VMEM((_CH, _D), jnp.float32),
            pltpu.SemaphoreType.DMA,
            pltpu.SemaphoreType.DMA,
            pltpu.SemaphoreType.DMA,
            pltpu.SemaphoreType.DMA,
        ],
    )(_body)
    out = run(flat_ids, entity_tokens, entity_counts,
              word_embeds.astype(jnp.bfloat16), idf)
    return out.reshape(b, l, _D)


# speedup vs baseline: 1.4675x; 1.0100x over previous
import functools

import jax
import jax.numpy as jnp
from jax import lax
from jax.experimental import pallas as pl
from jax.experimental.pallas import tpu as pltpu
from jax.experimental.pallas import tpu_sc as plsc

_NUM_WORDS = 100000
_T = 32
_D = 64
_LANES = 16

_NC = 2
_NS = 16
_NW = _NC * _NS

_CH = 8
_HF = 4
_HT = _HF * _T


def _body(ids_hbm, toks_hbm, cnts_hbm, emb_hbm, idf_hbm, out_hbm,
          idf_v, ids_v, toks2, cnts2, idxf, emb2, out_v,
          sem_t0, sem_t1, gsem0, gsem1, gsem2, gsem3):
    qpw = ids_v.shape[0] - 2 * _CH
    n_chunks = qpw // _CH
    wid = lax.axis_index("s") * _NC + lax.axis_index("c")
    base = wid * qpw

    pltpu.sync_copy(idf_hbm, idf_v)
    ids_v[pl.ds(qpw, 2 * _CH)] = jnp.zeros((2 * _CH,), jnp.int32)
    pltpu.sync_copy(ids_hbm.at[pl.ds(base, qpw)], ids_v.at[pl.ds(0, qpw)])

    def fire_toks(c, sl, sem):
        idx = ids_v.at[pl.ds(c * _CH, _CH)]
        cp_t = pltpu.async_copy(toks_hbm.at[idx], toks2.at[sl], sem)
        cp_c = pltpu.async_copy(cnts_hbm.at[idx], cnts2.at[sl], sem)
        return cp_t, cp_c

    def wait_toks(sem):
        pltpu.make_async_copy(toks_hbm.at[pl.ds(0, _CH)],
                              toks2.at[0], sem).wait()
        pltpu.make_async_copy(cnts_hbm.at[pl.ds(0, _CH)],
                              cnts2.at[0], sem).wait()

    gsems = (gsem0, gsem1, gsem2, gsem3)

    def fire_emb(sl, par, hoff, slot):
        for jj in range(_HF):
            for h in range(2):
                idxf[par, hoff, pl.ds(jj * _T + h * _LANES, _LANES)] = \
                    toks2[sl, hoff * _HF + jj, pl.ds(h * _LANES, _LANES)]
        pltpu.async_copy(emb_hbm.at[idxf.at[par, hoff]], emb2.at[slot],
                         gsems[slot])

    def drain_emb(slot):
        pltpu.make_async_copy(emb_hbm.at[pl.ds(0, _HT)],
                              emb2.at[slot], gsems[slot]).wait()

    def compute4(sl, qoff, slot):
        for jj in range(_HF):
            q = qoff + jj
            t0 = toks2[sl, q, pl.ds(0, _LANES)]
            t1 = toks2[sl, q, pl.ds(_LANES, _LANES)]
            w0 = cnts2[sl, q, pl.ds(0, _LANES)] * plsc.load_gather(idf_v, [t0])
            w1 = cnts2[sl, q, pl.ds(_LANES, _LANES)] * plsc.load_gather(idf_v, [t1])
            denom = jnp.maximum(jnp.sum(w0 + w1), 1e-6)
            scale = 1.0 / jnp.broadcast_to(denom, (_LANES,))

            accs = [jnp.zeros((_LANES,), jnp.float32) for _ in range(4)]
            hi_mask = jnp.full((_LANES,), -65536, jnp.int32)
            for half, wv in enumerate((w0, w1)):
                for tt in range(_LANES):
                    t = half * _LANES + tt
                    wt = jnp.broadcast_to(wv[tt], (_LANES,))
                    for blk in range(2):
                        v = plsc.bitcast(
                            emb2[slot, jj * _T + t, pl.ds(blk * 2 * _LANES, 2 * _LANES)],
                            jnp.int32)
                        lo = plsc.bitcast(v << 16, jnp.float32)
                        hi = plsc.bitcast(v & hi_mask, jnp.float32)
                        accs[2 * blk] = accs[2 * blk] + wt * lo
                        accs[2 * blk + 1] = accs[2 * blk + 1] + wt * hi
            lane2 = 2 * lax.iota(jnp.int32, _LANES)
            qvec = jnp.full((_LANES,), q, jnp.int32)
            for blk in range(2):
                plsc.store_scatter(out_v, [qvec, lane2 + 32 * blk],
                                   accs[2 * blk] * scale)
                plsc.store_scatter(out_v, [qvec, lane2 + 32 * blk + 1],
                                   accs[2 * blk + 1] * scale)

    cp_t, cp_c = fire_toks(0, 0, sem_t0)
    cp_t.wait()
    cp_c.wait()
    fire_toks(1, 1, sem_t1)
    fire_emb(0, 0, 0, 0)
    fire_emb(0, 0, 1, 1)

    sems = (sem_t0, sem_t1)

    @pl.loop(0, n_chunks // 2)
    def _pair(cp):
        for p in range(2):
            c = cp * 2 + p
            slc = lax.rem(c, 4)
            sln = lax.rem(c + 1, 4)
            my_a, my_b = 2 * p, 2 * p + 1
            nx_a, nx_b = (2 * p + 2) % 4, (2 * p + 3) % 4
            fire_toks(c + 2, lax.rem(c + 2, 4), sems[p])
            wait_toks(sems[1 - p])
            fire_emb(sln, 1 - p, 0, nx_a)
            drain_emb(my_a)
            compute4(slc, 0, my_a)
            fire_emb(sln, 1 - p, 1, nx_b)
            drain_emb(my_b)
            compute4(slc, _HF, my_b)
            pltpu.sync_copy(out_v, out_hbm.at[pl.ds(base + c * _CH, _CH)])

    drain_emb(0)
    drain_emb(1)
    wait_toks(sem_t1)


def kernel(entity_id, entity_tokens, entity_counts, word_embeds, idf):
    b, l = entity_id.shape
    q = b * l
    qpw = q // _NW
    flat_ids = entity_id.reshape(q).astype(jnp.int32)

    mesh = plsc.VectorSubcoreMesh(core_axis_name="c", subcore_axis_name="s")
    run = functools.partial(
        pl.kernel,
        out_type=jax.ShapeDtypeStruct((q, _D), jnp.float32),
        mesh=mesh,
        compiler_params=pltpu.CompilerParams(
            needs_layout_passes=False, use_tc_tiling_on_sc=False),
        scratch_types=[
            pltpu.VMEM((_NUM_WORDS,), jnp.float32),
            pltpu.VMEM((qpw + 2 * _CH,), jnp.int32),
            pltpu.VMEM((4, _CH, _T), jnp.int32),
            pltpu.VMEM((4, _CH, _T), jnp.float32),
            pltpu.VMEM((2, 2, _HT), jnp.int32),
            pltpu.VMEM((4, _HT, _D), jnp.bfloat16),
            pltpu.VMEM((_CH, _D), jnp.float32),
            pltpu.SemaphoreType.DMA,
            pltpu.SemaphoreType.DMA,
            pltpu.SemaphoreType.DMA,
            pltpu.SemaphoreType.DMA,
            pltpu.SemaphoreType.DMA,
            pltpu.SemaphoreType.DMA,
        ],
    )(_body)
    out = run(flat_ids, entity_tokens, entity_counts,
              word_embeds.astype(jnp.bfloat16), idf)
    return out.reshape(b, l, _D)
